# baseline (speedup 1.0000x reference)
import jax
import jax.numpy as jnp
from jax import lax
from jax.experimental import pallas as pl
from jax.experimental.pallas import tpu as pltpu
from jax.experimental.pallas import tpu_sc as plsc

NUM_HEADS = 8
NUM_BINS = 16
N_EDGES_INNER = NUM_BINS - 1
E_TOTAL = 3200000
NC, NS, L = 2, 16, 16
NW = NC * NS
CHUNK = 3200
NCHUNKS = E_TOTAL // CHUNK
ROUNDS = -(-NCHUNKS // NW)
TILES = CHUNK // 128
GPT = 128 // L


def _body(z_hbm, edges_hbm, tablet_hbm, out_hbm, zbuf, obuf, table_v, edges_v,
          sin0, sin1, sout0, sout1):
    wid = lax.axis_index("s") * NC + lax.axis_index("c")

    pltpu.sync_copy(tablet_hbm, table_v)
    pltpu.sync_copy(edges_hbm, edges_v)

    edges = [
        plsc.load_gather(edges_v, [jnp.full((L,), i + 1, jnp.int32)])
        for i in range(N_EDGES_INNER)
    ]
    gpat = [jnp.full((L,), h * NUM_BINS, jnp.int32) for h in range(NUM_HEADS)]
    sin = (sin0, sin1)
    sout = (sout0, sout1)

    def start_in(t, s):
        @pl.when(t < NCHUNKS)
        def _():
            pltpu.async_copy(z_hbm.at[pl.ds(t * CHUNK, CHUNK)], zbuf.at[s], sin[s])

    def wait_in(t, s):
        @pl.when(t < NCHUNKS)
        def _():
            pltpu.make_async_copy(
                z_hbm.at[pl.ds(t * CHUNK, CHUNK)], zbuf.at[s], sin[s]
            ).wait()

    def start_out(t, s):
        @pl.when(t < NCHUNKS)
        def _():
            pltpu.async_copy(obuf.at[s], out_hbm.at[pl.ds(t * TILES, TILES)], sout[s])

    def wait_out(t, s):
        @pl.when((t >= 0) & (t < NCHUNKS))
        def _():
            pltpu.make_async_copy(
                obuf.at[s], out_hbm.at[pl.ds(t * TILES, TILES)], sout[s]
            ).wait()

    def compute(t, s):
        @pl.when(t < NCHUNKS)
        def _():
            def tile_body(tt, _):
                for u in range(GPT):
                    off = tt * 128 + u * L
                    z = zbuf[s, pl.ds(off, L)]
                    one = jnp.ones((L,), jnp.int32)
                    zero = jnp.zeros((L,), jnp.int32)
                    terms = [jnp.where(z > e, one, zero) for e in edges]
                    while len(terms) > 1:
                        terms = [
                            terms[i] + terms[i + 1] if i + 1 < len(terms) else terms[i]
                            for i in range(0, len(terms), 2)
                        ]
                    bin_idx = terms[0]
                    for h in range(NUM_HEADS):
                        vals = plsc.load_gather(table_v, [bin_idx + gpat[h]])
                        obuf[s, tt, h, pl.ds(u * L, L)] = vals
                return 0

            lax.fori_loop(0, TILES, tile_body, 0)

    start_in(wid, 0)

    def pair_body(k, _):
        tA = 2 * k * NW + wid
        tB = tA + NW
        tC = tB + NW
        start_in(tB, 1)
        wait_in(tA, 0)
        wait_out(tA - 2 * NW, 0)
        compute(tA, 0)
        start_out(tA, 0)
        start_in(tC, 0)
        wait_in(tB, 1)
        wait_out(tB - 2 * NW, 1)
        compute(tB, 1)
        start_out(tB, 1)
        return 0

    lax.fori_loop(0, ROUNDS // 2, pair_body, 0)
    wait_out((ROUNDS - 2) * NW + wid, 0)
    wait_out((ROUNDS - 1) * NW + wid, 1)


def kernel(z_vals, bin_edges, table):
    edges16 = jnp.pad(bin_edges[1:NUM_BINS], (1, 0))
    tablet = table.T.reshape(-1)

    mesh = plsc.VectorSubcoreMesh(core_axis_name="c", subcore_axis_name="s")
    out3 = pl.kernel(
        _body,
        out_type=jax.ShapeDtypeStruct((E_TOTAL // 128, NUM_HEADS, 128), jnp.float32),
        mesh=mesh,
        compiler_params=pltpu.CompilerParams(needs_layout_passes=False),
        scratch_types=[
            pltpu.VMEM((2, CHUNK), jnp.float32),
            pltpu.VMEM((2, TILES, NUM_HEADS, 128), jnp.float32),
            pltpu.VMEM((NUM_BINS * NUM_HEADS,), jnp.float32),
            pltpu.VMEM((L,), jnp.float32),
            pltpu.SemaphoreType.DMA,
            pltpu.SemaphoreType.DMA,
            pltpu.SemaphoreType.DMA,
            pltpu.SemaphoreType.DMA,
        ],
    )(z_vals, edges16, tablet)
    return out3.transpose(0, 2, 1).reshape(E_TOTAL, NUM_HEADS)


# speedup vs baseline: 2.4938x; 2.4938x over previous
import jax
import jax.numpy as jnp
from jax import lax
from jax.experimental import pallas as pl
from jax.experimental.pallas import tpu as pltpu
from jax.experimental.pallas import tpu_sc as plsc

NUM_HEADS = 8
NUM_BINS = 16
N_EDGES_INNER = NUM_BINS - 1
E_TOTAL = 3200000
NC, NS, L = 2, 16, 16
NW = NC * NS
CHUNK = 3200
NCHUNKS = E_TOTAL // CHUNK
ROUNDS = -(-NCHUNKS // NW)
TILES = CHUNK // 128
GPT = 128 // L


def _body(z_hbm, edges_hbm, tablet_hbm, out_hbm, zbuf, obuf, table_v, edges_v,
          sin0, sin1, sout0, sout1):
    wid = lax.axis_index("s") * NC + lax.axis_index("c")

    pltpu.sync_copy(tablet_hbm, table_v)
    pltpu.sync_copy(edges_hbm, edges_v)

    edges = [
        plsc.load_gather(edges_v, [jnp.full((L,), i + 1, jnp.int32)])
        for i in range(N_EDGES_INNER)
    ]
    gpat = [jnp.full((L,), h * NUM_BINS, jnp.int32) for h in range(NUM_HEADS)]
    sin = (sin0, sin1)
    sout = (sout0, sout1)

    def start_in(t, s):
        @pl.when(t < NCHUNKS)
        def _():
            pltpu.async_copy(z_hbm.at[pl.ds(t * CHUNK, CHUNK)], zbuf.at[s], sin[s])

    def wait_in(t, s):
        @pl.when(t < NCHUNKS)
        def _():
            pltpu.make_async_copy(
                z_hbm.at[pl.ds(t * CHUNK, CHUNK)], zbuf.at[s], sin[s]
            ).wait()

    def start_out(t, s):
        @pl.when(t < NCHUNKS)
        def _():
            pltpu.async_copy(obuf.at[s], out_hbm.at[pl.ds(t * TILES, TILES)], sout[s])

    def wait_out(t, s):
        @pl.when((t >= 0) & (t < NCHUNKS))
        def _():
            pltpu.make_async_copy(
                obuf.at[s], out_hbm.at[pl.ds(t * TILES, TILES)], sout[s]
            ).wait()

    def compute(t, s):
        @pl.when(t < NCHUNKS)
        def _():
            @plsc.parallel_loop(0, TILES)
            def tile_body(tt):
                for u in range(GPT):
                    off = tt * 128 + u * L
                    z = zbuf[s, pl.ds(off, L)]
                    one = jnp.ones((L,), jnp.int32)
                    zero = jnp.zeros((L,), jnp.int32)
                    terms = [jnp.where(z > e, one, zero) for e in edges]
                    while len(terms) > 1:
                        terms = [
                            terms[i] + terms[i + 1] if i + 1 < len(terms) else terms[i]
                            for i in range(0, len(terms), 2)
                        ]
                    bin_idx = terms[0]
                    for h in range(NUM_HEADS):
                        vals = plsc.load_gather(table_v, [bin_idx + gpat[h]])
                        obuf[s, tt, h, pl.ds(u * L, L)] = vals

    start_in(wid, 0)

    def pair_body(k, _):
        tA = 2 * k * NW + wid
        tB = tA + NW
        tC = tB + NW
        start_in(tB, 1)
        wait_in(tA, 0)
        wait_out(tA - 2 * NW, 0)
        compute(tA, 0)
        start_out(tA, 0)
        start_in(tC, 0)
        wait_in(tB, 1)
        wait_out(tB - 2 * NW, 1)
        compute(tB, 1)
        start_out(tB, 1)
        return 0

    lax.fori_loop(0, ROUNDS // 2, pair_body, 0)
    wait_out((ROUNDS - 2) * NW + wid, 0)
    wait_out((ROUNDS - 1) * NW + wid, 1)


def kernel(z_vals, bin_edges, table):
    edges16 = jnp.pad(bin_edges[1:NUM_BINS], (1, 0))
    tablet = table.T.reshape(-1)

    mesh = plsc.VectorSubcoreMesh(core_axis_name="c", subcore_axis_name="s")
    out3 = pl.kernel(
        _body,
        out_type=jax.ShapeDtypeStruct((E_TOTAL // 128, NUM_HEADS, 128), jnp.float32),
        mesh=mesh,
        compiler_params=pltpu.CompilerParams(needs_layout_passes=False),
        scratch_types=[
            pltpu.VMEM((2, CHUNK), jnp.float32),
            pltpu.VMEM((2, TILES, NUM_HEADS, 128), jnp.float32),
            pltpu.VMEM((NUM_BINS * NUM_HEADS,), jnp.float32),
            pltpu.VMEM((L,), jnp.float32),
            pltpu.SemaphoreType.DMA,
            pltpu.SemaphoreType.DMA,
            pltpu.SemaphoreType.DMA,
            pltpu.SemaphoreType.DMA,
        ],
    )(z_vals, edges16, tablet)
    return out3.transpose(0, 2, 1).reshape(E_TOTAL, NUM_HEADS)


# speedup vs baseline: 2.7787x; 1.1143x over previous
import jax
import jax.numpy as jnp
from jax import lax
from jax.experimental import pallas as pl
from jax.experimental.pallas import tpu as pltpu
from jax.experimental.pallas import tpu_sc as plsc

NUM_HEADS = 8
NUM_BINS = 16
N_EDGES_INNER = NUM_BINS - 1
E_TOTAL = 3200000
NC, NS, L = 2, 16, 16
NW = NC * NS
CHUNK = 6400
NCHUNKS = E_TOTAL // CHUNK
ROUNDS = -(-NCHUNKS // NW)
TILES = CHUNK // 128
GPT = 128 // L


def _body(z_hbm, edges_hbm, tablet_hbm, out_hbm, zbuf, obuf, table_v, edges_v,
          sin0, sin1, sout0, sout1):
    wid = lax.axis_index("s") * NC + lax.axis_index("c")

    pltpu.sync_copy(tablet_hbm, table_v)
    pltpu.sync_copy(edges_hbm, edges_v)

    edges = [
        plsc.load_gather(edges_v, [jnp.full((L,), i + 1, jnp.int32)])
        for i in range(N_EDGES_INNER)
    ]
    gpat = [jnp.full((L,), h * NUM_BINS, jnp.int32) for h in range(NUM_HEADS)]
    sin = (sin0, sin1)
    sout = (sout0, sout1)

    def start_in(t, s):
        @pl.when(t < NCHUNKS)
        def _():
            pltpu.async_copy(z_hbm.at[pl.ds(t * CHUNK, CHUNK)], zbuf.at[s], sin[s])

    def wait_in(t, s):
        @pl.when(t < NCHUNKS)
        def _():
            pltpu.make_async_copy(
                z_hbm.at[pl.ds(t * CHUNK, CHUNK)], zbuf.at[s], sin[s]
            ).wait()

    def start_out(t, s):
        @pl.when(t < NCHUNKS)
        def _():
            pltpu.async_copy(obuf.at[s], out_hbm.at[pl.ds(t * TILES, TILES)], sout[s])

    def wait_out(t, s):
        @pl.when((t >= 0) & (t < NCHUNKS))
        def _():
            pltpu.make_async_copy(
                obuf.at[s], out_hbm.at[pl.ds(t * TILES, TILES)], sout[s]
            ).wait()

    def compute(t, s):
        @pl.when(t < NCHUNKS)
        def _():
            @plsc.parallel_loop(0, TILES)
            def tile_body(tt):
                for u in range(GPT):
                    off = tt * 128 + u * L
                    z = zbuf[s, pl.ds(off, L)]
                    one = jnp.ones((L,), jnp.int32)
                    zero = jnp.zeros((L,), jnp.int32)
                    terms = [jnp.where(z > e, one, zero) for e in edges]
                    while len(terms) > 1:
                        terms = [
                            terms[i] + terms[i + 1] if i + 1 < len(terms) else terms[i]
                            for i in range(0, len(terms), 2)
                        ]
                    bin_idx = terms[0]
                    for h in range(NUM_HEADS):
                        vals = plsc.load_gather(table_v, [bin_idx + gpat[h]])
                        obuf[s, tt, h, pl.ds(u * L, L)] = vals

    start_in(wid, 0)

    def pair_body(k, _):
        tA = 2 * k * NW + wid
        tB = tA + NW
        tC = tB + NW
        start_in(tB, 1)
        wait_in(tA, 0)
        wait_out(tA - 2 * NW, 0)
        compute(tA, 0)
        start_out(tA, 0)
        start_in(tC, 0)
        wait_in(tB, 1)
        wait_out(tB - 2 * NW, 1)
        compute(tB, 1)
        start_out(tB, 1)
        return 0

    lax.fori_loop(0, ROUNDS // 2, pair_body, 0)
    wait_out((ROUNDS - 2) * NW + wid, 0)
    wait_out((ROUNDS - 1) * NW + wid, 1)


def kernel(z_vals, bin_edges, table):
    edges16 = jnp.pad(bin_edges[1:NUM_BINS], (1, 0))
    tablet = table.T.reshape(-1)

    mesh = plsc.VectorSubcoreMesh(core_axis_name="c", subcore_axis_name="s")
    out3 = pl.kernel(
        _body,
        out_type=jax.ShapeDtypeStruct((E_TOTAL // 128, NUM_HEADS, 128), jnp.float32),
        mesh=mesh,
        compiler_params=pltpu.CompilerParams(needs_layout_passes=False),
        scratch_types=[
            pltpu.VMEM((2, CHUNK), jnp.float32),
            pltpu.VMEM((2, TILES, NUM_HEADS, 128), jnp.float32),
            pltpu.VMEM((NUM_BINS * NUM_HEADS,), jnp.float32),
            pltpu.VMEM((L,), jnp.float32),
            pltpu.SemaphoreType.DMA,
            pltpu.SemaphoreType.DMA,
            pltpu.SemaphoreType.DMA,
            pltpu.SemaphoreType.DMA,
        ],
    )(z_vals, edges16, tablet)
    return out3.transpose(0, 2, 1).reshape(E_TOTAL, NUM_HEADS)
